# Initial kernel scaffold; baseline (speedup 1.0000x reference)
import jax
import jax.numpy as jnp
from jax.experimental import pallas as pl


def kernel(x, emb_weight):
    raise NotImplementedError("write your pallas kernel here")


# speedup vs baseline: 2.9898x; 2.9898x over previous
import jax
import jax.numpy as jnp
from jax.experimental import pallas as pl

_BLOCK_ROWS = 1024


def _copy_block(w_ref, o_ref):
    o_ref[...] = w_ref[...]


def kernel(x, emb_weight):
    sl = x.shape[1]
    rows, dim = emb_weight.shape
    assert sl == rows
    grid = rows // _BLOCK_ROWS
    return pl.pallas_call(
        _copy_block,
        grid=(grid,),
        in_specs=[pl.BlockSpec((_BLOCK_ROWS, dim), lambda i: (i, 0))],
        out_specs=pl.BlockSpec((_BLOCK_ROWS, dim), lambda i: (i, 0)),
        out_shape=jax.ShapeDtypeStruct((rows, dim), emb_weight.dtype),
    )(emb_weight)
